# baseline (speedup 1.0000x reference)
import functools

import jax
import jax.numpy as jnp
from jax import lax
from jax.experimental import pallas as pl
from jax.experimental.pallas import tpu as pltpu
from jax.experimental.pallas import tpu_sc as plsc

N = 10000
F_IN = 128
HID = 8
HEADS = 8
HH = HEADS * HID
OUT = 16

NP = 10016
E0 = 320000
ET = E0 + N
EP = 344064
NC, NS, L = 2, 16, 16
NW = NC * NS
NPV = NP // L

CA1 = 2048
EQ1 = EP // 4
NCH_A1 = EQ1 // CA1
EH = EP // 2
CB1 = 2048
NCH_B1 = EH // CB1
CB2 = 4096
NCH_B2 = EH // CB2
ES2 = EP // NW
CA2 = 1344
NCH_A2 = ES2 // CA2

f32 = jnp.float32
i32 = jnp.int32

_SC_PARAMS = pltpu.CompilerParams(needs_layout_passes=False)


@functools.cache
def _mesh():
    return plsc.VectorSubcoreMesh(core_axis_name="c", subcore_axis_name="s",
                                  num_cores=NC, num_subcores=NS)


def _wid():
    return lax.axis_index("s") * NC + lax.axis_index("c")


def _vmap_loop(ref_ops, n, unroll=4):
    @plsc.parallel_loop(0, n, unroll=unroll)
    def _(i):
        ref_ops(pl.ds(i * L, L))


def _tc1_body(x_ref, w1_ref, asbd_ref, adbd_ref,
              xpT_ref, asT_ref, adT_ref, b_ref):
    x = x_ref[...]
    xpT = lax.dot_general(w1_ref[...], x, (((1,), (1,)), ((), ())),
                          preferred_element_type=f32)
    xpT_ref[...] = xpT
    asT = jnp.dot(asbd_ref[...], xpT, preferred_element_type=f32)
    adT = jnp.dot(adbd_ref[...], xpT, preferred_element_type=f32)
    asT_ref[...] = asT
    adT_ref[...] = adT
    s = (jnp.max(asT, axis=1, keepdims=True)
         + jnp.max(adT, axis=1, keepdims=True))
    b = jnp.maximum(s, 0.2 * s)
    b_ref[...] = jnp.broadcast_to(b, (HEADS, 16))


_tc1 = pl.pallas_call(
    _tc1_body,
    out_shape=[
        jax.ShapeDtypeStruct((HH, NP), f32),
        jax.ShapeDtypeStruct((HEADS, NP), f32),
        jax.ShapeDtypeStruct((HEADS, NP), f32),
        jax.ShapeDtypeStruct((HEADS, 16), f32),
    ],
)


def _tc3_body(o1_ref, dp_ref, b1_ref, w2_ref, as2w_ref, ad2w_ref,
              hpT_ref, s2_ref, d2_ref, b2_ref):
    den = 0.5 * jnp.sum(dp_ref[...], axis=1)
    rden = 1.0 / (den + 1e-16)
    rden64 = jnp.broadcast_to(rden[:, None, :], (HEADS, HID, NP)).reshape(HH, NP)
    hsum = (o1_ref[0] + o1_ref[1]) * rden64 + b1_ref[...]
    h = jnp.where(hsum > 0, hsum, jnp.exp(jnp.minimum(hsum, 0.0)) - 1.0)
    hpT = jnp.dot(w2_ref[...], h, preferred_element_type=f32)
    hpT_ref[...] = hpT
    s2 = jnp.dot(as2w_ref[...], hpT, preferred_element_type=f32)
    d2 = jnp.dot(ad2w_ref[...], hpT, preferred_element_type=f32)
    s2_ref[...] = s2
    d2_ref[...] = d2
    t = (jnp.max(s2, axis=1, keepdims=True)
         + jnp.max(d2, axis=1, keepdims=True))
    b2 = jnp.maximum(t, 0.2 * t)
    b2_ref[...] = jnp.broadcast_to(b2, (1, 16))


_tc3 = pl.pallas_call(
    _tc3_body,
    out_shape=[
        jax.ShapeDtypeStruct((OUT, NP), f32),
        jax.ShapeDtypeStruct((1, NP), f32),
        jax.ShapeDtypeStruct((1, NP), f32),
        jax.ShapeDtypeStruct((1, 16), f32),
    ],
)


def _tc5_body(o2_ref, dp_ref, b2_ref, out_ref):
    den = 0.0625 * jnp.sum(dp_ref[...], axis=0, keepdims=True)
    rden = 1.0 / (den + 1e-16)
    out_ref[...] = (o2_ref[0] + o2_ref[1]) * rden + b2_ref[...]


_tc5 = pl.pallas_call(
    _tc5_body,
    out_shape=[jax.ShapeDtypeStruct((OUT, NP), f32)],
)


def _sc_l1_body(as_hbm, ad_hbm, xp_hbm, src_hbm, dst_hbm, bv_hbm,
                o1_hbm, dp_hbm,
                as_v, ad_v, den_v, bv_v,
                acc0, acc1, acc2, acc3, xp0, xp1, xp2, xp3,
                src0, src1, dst0, dst1,
                ss0, ss1, sd0, sd1):
    wid = _wid()
    cq = wid % 16
    hf = wid // 16
    h = cq // 2
    q = (cq % 2) * 2 + hf
    accs = (acc0, acc1, acc2, acc3)
    xps = (xp0, xp1, xp2, xp3)
    srcs, dsts = (src0, src1), (dst0, dst1)
    sss, sds = (ss0, ss1), (sd0, sd1)
    base = hf * EH

    def start_in(j, b):
        off = pl.multiple_of(base + j * CB1, 8)
        pltpu.async_copy(src_hbm.at[pl.ds(off, CB1)], srcs[b], sss[b])
        pltpu.async_copy(dst_hbm.at[pl.ds(off, CB1)], dsts[b], sds[b])

    def wait_in(b):
        pltpu.make_async_copy(src_hbm.at[pl.ds(0, CB1)], srcs[b], sss[b]).wait()
        pltpu.make_async_copy(dst_hbm.at[pl.ds(0, CB1)], dsts[b], sds[b]).wait()

    start_in(0, 0)

    pltpu.sync_copy(as_hbm.at[h], as_v)
    pltpu.sync_copy(ad_hbm.at[h], ad_v)
    pltpu.sync_copy(bv_hbm, bv_v)
    bh = bv_v[h, :]

    def zero(sl):
        den_v[sl] = jnp.zeros((L,), f32)
    _vmap_loop(zero, NPV)

    for k in range(4):
        pltpu.sync_copy(xp_hbm.at[4 * cq + k], xps[k])

        def zeroa(sl, a=accs[k]):
            a[sl] = jnp.zeros((L,), f32)
        _vmap_loop(zeroa, NPV)

    def outer(g, _):
        for b in range(2):
            j = 2 * g + b
            wait_in(b)
            if b == 0:
                start_in(j + 1, 1)
            else:
                @pl.when(g < NCH_B1 // 2 - 1)
                def _():
                    start_in(j + 1, 0)

            def inner(sl, b=b):
                si = srcs[b][sl]
                di = dsts[b][sl]
                al = plsc.load_gather(as_v, [si]) + plsc.load_gather(ad_v, [di])
                al = jnp.maximum(al, 0.2 * al)
                e = jnp.exp(al - bh)
                plsc.addupdate_scatter(den_v, [di], e)
                for k in range(4):
                    xv = plsc.load_gather(xps[k], [si])
                    plsc.addupdate_scatter(accs[k], [di], e * xv)
            _vmap_loop(inner, CB1 // L, unroll=4)
        return 0
    lax.fori_loop(0, NCH_B1 // 2, outer, 0)
    for k in range(4):
        pltpu.sync_copy(accs[k], o1_hbm.at[hf, 4 * cq + k])
    pltpu.sync_copy(den_v, dp_hbm.at[h, q])


@functools.cache
def _sc_l1():
  return pl.kernel(
    _sc_l1_body,
    out_type=[
        jax.ShapeDtypeStruct((2, HH, NP), f32),
        jax.ShapeDtypeStruct((HEADS, 4, NP), f32),
    ],
    mesh=_mesh(),
    compiler_params=_SC_PARAMS,
    scratch_types=[
        pltpu.VMEM((NP,), f32),
        pltpu.VMEM((NP,), f32),
        pltpu.VMEM((NP,), f32),
        pltpu.VMEM((HEADS, 16), f32),
        pltpu.VMEM((NP,), f32),
        pltpu.VMEM((NP,), f32),
        pltpu.VMEM((NP,), f32),
        pltpu.VMEM((NP,), f32),
        pltpu.VMEM((NP,), f32),
        pltpu.VMEM((NP,), f32),
        pltpu.VMEM((NP,), f32),
        pltpu.VMEM((NP,), f32),
        pltpu.VMEM((CB1,), i32),
        pltpu.VMEM((CB1,), i32),
        pltpu.VMEM((CB1,), i32),
        pltpu.VMEM((CB1,), i32),
        pltpu.SemaphoreType.DMA,
        pltpu.SemaphoreType.DMA,
        pltpu.SemaphoreType.DMA,
        pltpu.SemaphoreType.DMA,
    ],
  )


def _sc_l2_body(s2_hbm, d2_hbm, hp_hbm, src_hbm, dst_hbm, bv_hbm,
                o2_hbm, dp_hbm,
                s2_v, d2_v, den_v, bv_v, acc_v, hp_v,
                src0, src1, dst0, dst1,
                ss0, ss1, sd0, sd1):
    wid = _wid()
    ch = wid % 16
    hf = wid // 16
    srcs, dsts = (src0, src1), (dst0, dst1)
    sss, sds = (ss0, ss1), (sd0, sd1)
    base = hf * EH

    def start_in(j, b):
        off = pl.multiple_of(base + j * CB2, 8)
        pltpu.async_copy(src_hbm.at[pl.ds(off, CB2)], srcs[b], sss[b])
        pltpu.async_copy(dst_hbm.at[pl.ds(off, CB2)], dsts[b], sds[b])

    def wait_in(b):
        pltpu.make_async_copy(src_hbm.at[pl.ds(0, CB2)], srcs[b], sss[b]).wait()
        pltpu.make_async_copy(dst_hbm.at[pl.ds(0, CB2)], dsts[b], sds[b]).wait()

    start_in(0, 0)

    pltpu.sync_copy(s2_hbm.at[0], s2_v)
    pltpu.sync_copy(d2_hbm.at[0], d2_v)
    pltpu.sync_copy(hp_hbm.at[ch], hp_v)
    pltpu.sync_copy(bv_hbm, bv_v)
    bh = bv_v[0, :]

    def zero(sl):
        den_v[sl] = jnp.zeros((L,), f32)
        acc_v[sl] = jnp.zeros((L,), f32)
    _vmap_loop(zero, NPV)

    def outer(g, _):
        for b in range(2):
            j = 2 * g + b
            wait_in(b)
            if b == 0:
                start_in(j + 1, 1)
            else:
                @pl.when(g < NCH_B2 // 2 - 1)
                def _():
                    start_in(j + 1, 0)

            def inner(sl, b=b):
                si = srcs[b][sl]
                di = dsts[b][sl]
                al = plsc.load_gather(s2_v, [si]) + plsc.load_gather(d2_v, [di])
                al = jnp.maximum(al, 0.2 * al)
                e = jnp.exp(al - bh)
                plsc.addupdate_scatter(den_v, [di], e)
                xv = plsc.load_gather(hp_v, [si])
                plsc.addupdate_scatter(acc_v, [di], e * xv)
            _vmap_loop(inner, CB2 // L, unroll=4)
        return 0
    lax.fori_loop(0, NCH_B2 // 2, outer, 0)
    pltpu.sync_copy(acc_v, o2_hbm.at[hf, ch])
    pltpu.sync_copy(den_v, dp_hbm.at[wid])


@functools.cache
def _sc_l2():
  return pl.kernel(
    _sc_l2_body,
    out_type=[
        jax.ShapeDtypeStruct((2, OUT, NP), f32),
        jax.ShapeDtypeStruct((NW, NP), f32),
    ],
    mesh=_mesh(),
    compiler_params=_SC_PARAMS,
    scratch_types=[
        pltpu.VMEM((NP,), f32),
        pltpu.VMEM((NP,), f32),
        pltpu.VMEM((NP,), f32),
        pltpu.VMEM((1, 16), f32),
        pltpu.VMEM((NP,), f32),
        pltpu.VMEM((NP,), f32),
        pltpu.VMEM((CB2,), i32),
        pltpu.VMEM((CB2,), i32),
        pltpu.VMEM((CB2,), i32),
        pltpu.VMEM((CB2,), i32),
        pltpu.SemaphoreType.DMA,
        pltpu.SemaphoreType.DMA,
        pltpu.SemaphoreType.DMA,
        pltpu.SemaphoreType.DMA,
    ],
  )


def kernel(x, edge_index, W1, att_src1, att_dst1, b1, W2, att_src2, att_dst2, b2):
    x_p = jnp.pad(x, ((0, NP - N), (0, 0)))
    loop = jnp.arange(N, dtype=i32)
    padv = jnp.full((EP - ET,), N, i32)
    src_p = jnp.concatenate([edge_index[0].astype(i32), loop, padv])
    dst_p = jnp.concatenate([edge_index[1].astype(i32), loop, padv])

    eye = jnp.eye(HEADS, dtype=f32)
    asbd = (eye[:, :, None] * att_src1[None, :, :]).reshape(HEADS, HH)
    adbd = (eye[:, :, None] * att_dst1[None, :, :]).reshape(HEADS, HH)

    xpT, asT, adT, b1v = _tc1(x_p, W1, asbd, adbd)
    o1, dp1 = _sc_l1()(asT, adT, xpT, src_p, dst_p, b1v)
    hpT, s2, d2, b2v = _tc3(o1, dp1, b1.reshape(HH, 1), W2, att_src2, att_dst2)
    o2, dp2 = _sc_l2()(s2, d2, hpT, src_p, dst_p, b2v)
    (outf,) = _tc5(o2, dp2, b2.reshape(OUT, 1))
    return outf[:, :N].T


# speedup vs baseline: 1.0185x; 1.0185x over previous
import functools

import jax
import jax.numpy as jnp
from jax import lax
from jax.experimental import pallas as pl
from jax.experimental.pallas import tpu as pltpu
from jax.experimental.pallas import tpu_sc as plsc

N = 10000
F_IN = 128
HID = 8
HEADS = 8
HH = HEADS * HID
OUT = 16

NP = 10016
E0 = 320000
ET = E0 + N
EP = 344064
NC, NS, L = 2, 16, 16
NW = NC * NS
NPV = NP // L

CA1 = 2048
EQ1 = EP // 4
NCH_A1 = EQ1 // CA1
EH = EP // 2
CB1 = 2048
NCH_B1 = EH // CB1
CB2 = 4096
NCH_B2 = EH // CB2
ES2 = EP // NW
CA2 = 1344
NCH_A2 = ES2 // CA2

f32 = jnp.float32
i32 = jnp.int32

_SC_PARAMS = pltpu.CompilerParams(needs_layout_passes=False)


@functools.cache
def _mesh():
    return plsc.VectorSubcoreMesh(core_axis_name="c", subcore_axis_name="s",
                                  num_cores=NC, num_subcores=NS)


def _wid():
    return lax.axis_index("s") * NC + lax.axis_index("c")


def _vmap_loop(ref_ops, n, unroll=4):
    @plsc.parallel_loop(0, n, unroll=unroll)
    def _(i):
        ref_ops(pl.ds(i * L, L))


def _tc1_body(x_ref, w1lo_ref, w1hi_ref, asbd_lo_ref, asbd_hi_ref,
              adbd_lo_ref, adbd_hi_ref,
              xpp_ref, asT_ref, adT_ref, b_ref):
    x = x_ref[...]
    dn = (((1,), (1,)), ((), ()))
    xpTlo = lax.dot_general(w1lo_ref[...], x, dn,
                            preferred_element_type=f32)
    xpThi = lax.dot_general(w1hi_ref[...], x, dn,
                            preferred_element_type=f32)
    lo_w = lax.bitcast_convert_type(xpTlo.astype(jnp.bfloat16),
                                    jnp.uint16).astype(jnp.uint32)
    hi_w = lax.bitcast_convert_type(xpThi.astype(jnp.bfloat16),
                                    jnp.uint16).astype(jnp.uint32)
    xpp_ref[...] = lax.bitcast_convert_type(lo_w | (hi_w << 16), jnp.int32)
    asT = (jnp.dot(asbd_lo_ref[...], xpTlo, preferred_element_type=f32)
           + jnp.dot(asbd_hi_ref[...], xpThi, preferred_element_type=f32))
    adT = (jnp.dot(adbd_lo_ref[...], xpTlo, preferred_element_type=f32)
           + jnp.dot(adbd_hi_ref[...], xpThi, preferred_element_type=f32))
    asT_ref[...] = asT
    adT_ref[...] = adT
    s = (jnp.max(asT, axis=1, keepdims=True)
         + jnp.max(adT, axis=1, keepdims=True))
    b = jnp.maximum(s, 0.2 * s)
    b_ref[...] = jnp.broadcast_to(b, (HEADS, 16))


_tc1 = pl.pallas_call(
    _tc1_body,
    out_shape=[
        jax.ShapeDtypeStruct((HH // 2, NP), i32),
        jax.ShapeDtypeStruct((HEADS, NP), f32),
        jax.ShapeDtypeStruct((HEADS, NP), f32),
        jax.ShapeDtypeStruct((HEADS, 16), f32),
    ],
)


def _tc3_body(o1_ref, dp_ref, b1_ref, w2_ref, as2w_ref, ad2w_ref,
              hpT_ref, s2_ref, d2_ref, b2_ref):
    den = 0.5 * jnp.sum(dp_ref[...], axis=1)
    rden = 1.0 / (den + 1e-16)
    rden64 = jnp.broadcast_to(rden[:, None, :], (HEADS, HID, NP)).reshape(HH, NP)
    hsum = (o1_ref[0] + o1_ref[1]) * rden64 + b1_ref[...]
    h = jnp.where(hsum > 0, hsum, jnp.exp(jnp.minimum(hsum, 0.0)) - 1.0)
    hpT = jnp.dot(w2_ref[...], h, preferred_element_type=f32)
    hpT_ref[...] = hpT
    s2 = jnp.dot(as2w_ref[...], hpT, preferred_element_type=f32)
    d2 = jnp.dot(ad2w_ref[...], hpT, preferred_element_type=f32)
    s2_ref[...] = s2
    d2_ref[...] = d2
    t = (jnp.max(s2, axis=1, keepdims=True)
         + jnp.max(d2, axis=1, keepdims=True))
    b2 = jnp.maximum(t, 0.2 * t)
    b2_ref[...] = jnp.broadcast_to(b2, (1, 16))


_tc3 = pl.pallas_call(
    _tc3_body,
    out_shape=[
        jax.ShapeDtypeStruct((OUT, NP), f32),
        jax.ShapeDtypeStruct((1, NP), f32),
        jax.ShapeDtypeStruct((1, NP), f32),
        jax.ShapeDtypeStruct((1, 16), f32),
    ],
)


def _tc5_body(o2_ref, dp_ref, b2_ref, out_ref):
    den = 0.0625 * jnp.sum(dp_ref[...], axis=0, keepdims=True)
    rden = 1.0 / (den + 1e-16)
    out_ref[...] = (o2_ref[0] + o2_ref[1]) * rden + b2_ref[...]


_tc5 = pl.pallas_call(
    _tc5_body,
    out_shape=[jax.ShapeDtypeStruct((OUT, NP), f32)],
)


def _sc_l1_body(as_hbm, ad_hbm, xp_hbm, src_hbm, dst_hbm, bv_hbm,
                o1_hbm, dp_hbm,
                as_v, ad_v, den_v, bv_v,
                acc0, acc1, acc2, acc3, xp0, xp1,
                src0, src1, dst0, dst1,
                ss0, ss1, sd0, sd1):
    wid = _wid()
    cq = wid % 16
    hf = wid // 16
    h = cq // 2
    q = (cq % 2) * 2 + hf
    accs = (acc0, acc1, acc2, acc3)
    xpps = (xp0, xp1)
    srcs, dsts = (src0, src1), (dst0, dst1)
    sss, sds = (ss0, ss1), (sd0, sd1)
    base = hf * EH

    def start_in(j, b):
        off = pl.multiple_of(base + j * CB1, 8)
        pltpu.async_copy(src_hbm.at[pl.ds(off, CB1)], srcs[b], sss[b])
        pltpu.async_copy(dst_hbm.at[pl.ds(off, CB1)], dsts[b], sds[b])

    def wait_in(b):
        pltpu.make_async_copy(src_hbm.at[pl.ds(0, CB1)], srcs[b], sss[b]).wait()
        pltpu.make_async_copy(dst_hbm.at[pl.ds(0, CB1)], dsts[b], sds[b]).wait()

    start_in(0, 0)

    pltpu.sync_copy(as_hbm.at[h], as_v)
    pltpu.sync_copy(ad_hbm.at[h], ad_v)
    pltpu.sync_copy(bv_hbm, bv_v)
    bh = bv_v[h, :]

    def zero(sl):
        den_v[sl] = jnp.zeros((L,), f32)
    _vmap_loop(zero, NPV)

    for k in range(2):
        pltpu.sync_copy(xp_hbm.at[2 * cq + k], xpps[k])
    for k in range(4):
        def zeroa(sl, a=accs[k]):
            a[sl] = jnp.zeros((L,), f32)
        _vmap_loop(zeroa, NPV)

    def outer(g, _):
        for b in range(2):
            j = 2 * g + b
            wait_in(b)
            if b == 0:
                start_in(j + 1, 1)
            else:
                @pl.when(g < NCH_B1 // 2 - 1)
                def _():
                    start_in(j + 1, 0)

            def inner(sl, b=b):
                si = srcs[b][sl]
                di = dsts[b][sl]
                al = plsc.load_gather(as_v, [si]) + plsc.load_gather(ad_v, [di])
                al = jnp.maximum(al, 0.2 * al)
                e = jnp.exp(al - bh)
                plsc.addupdate_scatter(den_v, [di], e)
                for k in range(2):
                    gi = plsc.load_gather(xpps[k], [si])
                    pb = plsc.bitcast(gi, jnp.bfloat16)
                    lo, hi = plsc.unpack(pb, format=plsc.PackFormat.INTERLEAVED,
                                         preferred_element_type=f32)
                    plsc.addupdate_scatter(accs[2 * k], [di], e * lo)
                    plsc.addupdate_scatter(accs[2 * k + 1], [di], e * hi)
            _vmap_loop(inner, CB1 // L, unroll=4)
        return 0
    lax.fori_loop(0, NCH_B1 // 2, outer, 0)
    for k in range(4):
        pltpu.sync_copy(accs[k], o1_hbm.at[hf, 4 * cq + k])
    pltpu.sync_copy(den_v, dp_hbm.at[h, q])


@functools.cache
def _sc_l1():
  return pl.kernel(
    _sc_l1_body,
    out_type=[
        jax.ShapeDtypeStruct((2, HH, NP), f32),
        jax.ShapeDtypeStruct((HEADS, 4, NP), f32),
    ],
    mesh=_mesh(),
    compiler_params=_SC_PARAMS,
    scratch_types=[
        pltpu.VMEM((NP,), f32),
        pltpu.VMEM((NP,), f32),
        pltpu.VMEM((NP,), f32),
        pltpu.VMEM((HEADS, 16), f32),
        pltpu.VMEM((NP,), f32),
        pltpu.VMEM((NP,), f32),
        pltpu.VMEM((NP,), f32),
        pltpu.VMEM((NP,), f32),
        pltpu.VMEM((NP,), i32),
        pltpu.VMEM((NP,), i32),
        pltpu.VMEM((CB1,), i32),
        pltpu.VMEM((CB1,), i32),
        pltpu.VMEM((CB1,), i32),
        pltpu.VMEM((CB1,), i32),
        pltpu.SemaphoreType.DMA,
        pltpu.SemaphoreType.DMA,
        pltpu.SemaphoreType.DMA,
        pltpu.SemaphoreType.DMA,
    ],
  )


def _sc_l2_body(s2_hbm, d2_hbm, hp_hbm, src_hbm, dst_hbm, bv_hbm,
                o2_hbm, dp_hbm,
                s2_v, d2_v, den_v, bv_v, acc_v, hp_v,
                src0, src1, dst0, dst1,
                ss0, ss1, sd0, sd1):
    wid = _wid()
    ch = wid % 16
    hf = wid // 16
    srcs, dsts = (src0, src1), (dst0, dst1)
    sss, sds = (ss0, ss1), (sd0, sd1)
    base = hf * EH

    def start_in(j, b):
        off = pl.multiple_of(base + j * CB2, 8)
        pltpu.async_copy(src_hbm.at[pl.ds(off, CB2)], srcs[b], sss[b])
        pltpu.async_copy(dst_hbm.at[pl.ds(off, CB2)], dsts[b], sds[b])

    def wait_in(b):
        pltpu.make_async_copy(src_hbm.at[pl.ds(0, CB2)], srcs[b], sss[b]).wait()
        pltpu.make_async_copy(dst_hbm.at[pl.ds(0, CB2)], dsts[b], sds[b]).wait()

    start_in(0, 0)

    pltpu.sync_copy(s2_hbm.at[0], s2_v)
    pltpu.sync_copy(d2_hbm.at[0], d2_v)
    pltpu.sync_copy(hp_hbm.at[ch], hp_v)
    pltpu.sync_copy(bv_hbm, bv_v)
    bh = bv_v[0, :]

    def zero(sl):
        den_v[sl] = jnp.zeros((L,), f32)
        acc_v[sl] = jnp.zeros((L,), f32)
    _vmap_loop(zero, NPV)

    def outer(g, _):
        for b in range(2):
            j = 2 * g + b
            wait_in(b)
            if b == 0:
                start_in(j + 1, 1)
            else:
                @pl.when(g < NCH_B2 // 2 - 1)
                def _():
                    start_in(j + 1, 0)

            def inner(sl, b=b):
                si = srcs[b][sl]
                di = dsts[b][sl]
                al = plsc.load_gather(s2_v, [si]) + plsc.load_gather(d2_v, [di])
                al = jnp.maximum(al, 0.2 * al)
                e = jnp.exp(al - bh)
                plsc.addupdate_scatter(den_v, [di], e)
                xv = plsc.load_gather(hp_v, [si])
                plsc.addupdate_scatter(acc_v, [di], e * xv)
            _vmap_loop(inner, CB2 // L, unroll=4)
        return 0
    lax.fori_loop(0, NCH_B2 // 2, outer, 0)
    pltpu.sync_copy(acc_v, o2_hbm.at[hf, ch])
    pltpu.sync_copy(den_v, dp_hbm.at[wid])


@functools.cache
def _sc_l2():
  return pl.kernel(
    _sc_l2_body,
    out_type=[
        jax.ShapeDtypeStruct((2, OUT, NP), f32),
        jax.ShapeDtypeStruct((NW, NP), f32),
    ],
    mesh=_mesh(),
    compiler_params=_SC_PARAMS,
    scratch_types=[
        pltpu.VMEM((NP,), f32),
        pltpu.VMEM((NP,), f32),
        pltpu.VMEM((NP,), f32),
        pltpu.VMEM((1, 16), f32),
        pltpu.VMEM((NP,), f32),
        pltpu.VMEM((NP,), f32),
        pltpu.VMEM((CB2,), i32),
        pltpu.VMEM((CB2,), i32),
        pltpu.VMEM((CB2,), i32),
        pltpu.VMEM((CB2,), i32),
        pltpu.SemaphoreType.DMA,
        pltpu.SemaphoreType.DMA,
        pltpu.SemaphoreType.DMA,
        pltpu.SemaphoreType.DMA,
    ],
  )


def kernel(x, edge_index, W1, att_src1, att_dst1, b1, W2, att_src2, att_dst2, b2):
    x_p = jnp.pad(x, ((0, NP - N), (0, 0)))
    loop = jnp.arange(N, dtype=i32)
    padv = jnp.full((EP - ET,), N, i32)
    src_p = jnp.concatenate([edge_index[0].astype(i32), loop, padv])
    dst_p = jnp.concatenate([edge_index[1].astype(i32), loop, padv])

    eye = jnp.eye(HEADS, dtype=f32)
    asbd = (eye[:, :, None] * att_src1[None, :, :]).reshape(HEADS, HH)
    adbd = (eye[:, :, None] * att_dst1[None, :, :]).reshape(HEADS, HH)

    xpp, asT, adT, b1v = _tc1(x_p, W1[0::2], W1[1::2], asbd[:, 0::2],
                              asbd[:, 1::2], adbd[:, 0::2], adbd[:, 1::2])
    o1, dp1 = _sc_l1()(asT, adT, xpp, src_p, dst_p, b1v)
    hpT, s2, d2, b2v = _tc3(o1, dp1, b1.reshape(HH, 1), W2, att_src2, att_dst2)
    o2, dp2 = _sc_l2()(s2, d2, hpT, src_p, dst_p, b2v)
    (outf,) = _tc5(o2, dp2, b2.reshape(OUT, 1))
    return outf[:, :N].T


# speedup vs baseline: 1.0466x; 1.0275x over previous
import functools

import jax
import jax.numpy as jnp
from jax import lax
from jax.experimental import pallas as pl
from jax.experimental.pallas import tpu as pltpu
from jax.experimental.pallas import tpu_sc as plsc

N = 10000
F_IN = 128
HID = 8
HEADS = 8
HH = HEADS * HID
OUT = 16

NP = 10016
E0 = 320000
ET = E0 + N
EP = 344064
NC, NS, L = 2, 16, 16
NW = NC * NS
NPV = NP // L

CA1 = 2048
EQ1 = EP // 4
NCH_A1 = EQ1 // CA1
EH = EP // 2
CB1 = 2048
NCH_B1 = EH // CB1
EQ = EP // 4
CB2 = 2048
NCH_B2 = EQ // CB2
ES2 = EP // NW
CA2 = 1344
NCH_A2 = ES2 // CA2

f32 = jnp.float32
i32 = jnp.int32

_SC_PARAMS = pltpu.CompilerParams(needs_layout_passes=False)


@functools.cache
def _mesh():
    return plsc.VectorSubcoreMesh(core_axis_name="c", subcore_axis_name="s",
                                  num_cores=NC, num_subcores=NS)


def _wid():
    return lax.axis_index("s") * NC + lax.axis_index("c")


def _vmap_loop(ref_ops, n, unroll=4):
    @plsc.parallel_loop(0, n, unroll=unroll)
    def _(i):
        ref_ops(pl.ds(i * L, L))


def _tc1_body(x_ref, w1lo_ref, w1hi_ref, asbd_lo_ref, asbd_hi_ref,
              adbd_lo_ref, adbd_hi_ref,
              xpp_ref, asT_ref, adT_ref, b_ref):
    x = x_ref[...]
    dn = (((1,), (1,)), ((), ()))
    xpTlo = lax.dot_general(w1lo_ref[...], x, dn,
                            preferred_element_type=f32)
    xpThi = lax.dot_general(w1hi_ref[...], x, dn,
                            preferred_element_type=f32)
    lo_w = lax.bitcast_convert_type(xpTlo.astype(jnp.bfloat16),
                                    jnp.uint16).astype(jnp.uint32)
    hi_w = lax.bitcast_convert_type(xpThi.astype(jnp.bfloat16),
                                    jnp.uint16).astype(jnp.uint32)
    xpp_ref[...] = lax.bitcast_convert_type(lo_w | (hi_w << 16), jnp.int32)
    asT = (jnp.dot(asbd_lo_ref[...], xpTlo, preferred_element_type=f32)
           + jnp.dot(asbd_hi_ref[...], xpThi, preferred_element_type=f32))
    adT = (jnp.dot(adbd_lo_ref[...], xpTlo, preferred_element_type=f32)
           + jnp.dot(adbd_hi_ref[...], xpThi, preferred_element_type=f32))
    asT_ref[...] = asT
    adT_ref[...] = adT
    s = (jnp.max(asT, axis=1, keepdims=True)
         + jnp.max(adT, axis=1, keepdims=True))
    b = jnp.maximum(s, 0.2 * s)
    b_ref[...] = jnp.broadcast_to(b, (HEADS, 16))


_tc1 = pl.pallas_call(
    _tc1_body,
    out_shape=[
        jax.ShapeDtypeStruct((HH // 2, NP), i32),
        jax.ShapeDtypeStruct((HEADS, NP), f32),
        jax.ShapeDtypeStruct((HEADS, NP), f32),
        jax.ShapeDtypeStruct((HEADS, 16), f32),
    ],
)


def _tc3_body(o1_ref, dp_ref, b1_ref, w2lo_ref, w2hi_ref, as2wlo_ref,
              as2whi_ref, ad2wlo_ref, ad2whi_ref,
              hpp_ref, s2_ref, d2_ref, b2_ref):
    den = 0.5 * jnp.sum(dp_ref[...], axis=1)
    rden = 1.0 / (den + 1e-16)
    rden64 = jnp.broadcast_to(rden[:, None, :], (HEADS, HID, NP)).reshape(HH, NP)
    hsum = (o1_ref[0] + o1_ref[1]) * rden64 + b1_ref[...]
    h = jnp.where(hsum > 0, hsum, jnp.exp(jnp.minimum(hsum, 0.0)) - 1.0)
    hpTlo = jnp.dot(w2lo_ref[...], h, preferred_element_type=f32)
    hpThi = jnp.dot(w2hi_ref[...], h, preferred_element_type=f32)
    lo_w = lax.bitcast_convert_type(hpTlo.astype(jnp.bfloat16),
                                    jnp.uint16).astype(jnp.uint32)
    hi_w = lax.bitcast_convert_type(hpThi.astype(jnp.bfloat16),
                                    jnp.uint16).astype(jnp.uint32)
    hpp_ref[...] = lax.bitcast_convert_type(lo_w | (hi_w << 16), jnp.int32)
    s2 = (jnp.dot(as2wlo_ref[...], hpTlo, preferred_element_type=f32)
          + jnp.dot(as2whi_ref[...], hpThi, preferred_element_type=f32))
    d2 = (jnp.dot(ad2wlo_ref[...], hpTlo, preferred_element_type=f32)
          + jnp.dot(ad2whi_ref[...], hpThi, preferred_element_type=f32))
    s2_ref[...] = s2
    d2_ref[...] = d2
    t = (jnp.max(s2, axis=1, keepdims=True)
         + jnp.max(d2, axis=1, keepdims=True))
    b2 = jnp.maximum(t, 0.2 * t)
    b2_ref[...] = jnp.broadcast_to(b2, (1, 16))


_tc3 = pl.pallas_call(
    _tc3_body,
    out_shape=[
        jax.ShapeDtypeStruct((OUT // 2, NP), i32),
        jax.ShapeDtypeStruct((1, NP), f32),
        jax.ShapeDtypeStruct((1, NP), f32),
        jax.ShapeDtypeStruct((1, 16), f32),
    ],
)


def _tc5_body(o2_ref, dp_ref, b2_ref, out_ref):
    den = 0.125 * jnp.sum(dp_ref[...], axis=0, keepdims=True)
    rden = 1.0 / (den + 1e-16)
    osum = o2_ref[0] + o2_ref[1] + o2_ref[2] + o2_ref[3]
    out_ref[...] = osum * rden + b2_ref[...]


_tc5 = pl.pallas_call(
    _tc5_body,
    out_shape=[jax.ShapeDtypeStruct((OUT, NP), f32)],
)


def _sc_l1_body(as_hbm, ad_hbm, xp_hbm, src_hbm, dst_hbm, bv_hbm,
                o1_hbm, dp_hbm,
                as_v, ad_v, den_v, bv_v,
                acc0, acc1, acc2, acc3, xp0, xp1,
                src0, src1, dst0, dst1,
                ss0, ss1, sd0, sd1):
    wid = _wid()
    cq = wid % 16
    hf = wid // 16
    h = cq // 2
    q = (cq % 2) * 2 + hf
    accs = (acc0, acc1, acc2, acc3)
    xpps = (xp0, xp1)
    srcs, dsts = (src0, src1), (dst0, dst1)
    sss, sds = (ss0, ss1), (sd0, sd1)
    base = hf * EH

    def start_in(j, b):
        off = pl.multiple_of(base + j * CB1, 8)
        pltpu.async_copy(src_hbm.at[pl.ds(off, CB1)], srcs[b], sss[b])
        pltpu.async_copy(dst_hbm.at[pl.ds(off, CB1)], dsts[b], sds[b])

    def wait_in(b):
        pltpu.make_async_copy(src_hbm.at[pl.ds(0, CB1)], srcs[b], sss[b]).wait()
        pltpu.make_async_copy(dst_hbm.at[pl.ds(0, CB1)], dsts[b], sds[b]).wait()

    start_in(0, 0)

    pltpu.sync_copy(as_hbm.at[h], as_v)
    pltpu.sync_copy(ad_hbm.at[h], ad_v)
    pltpu.sync_copy(bv_hbm, bv_v)
    bh = bv_v[h, :]

    def zero(sl):
        den_v[sl] = jnp.zeros((L,), f32)
    _vmap_loop(zero, NPV)

    for k in range(2):
        pltpu.sync_copy(xp_hbm.at[2 * cq + k], xpps[k])
    for k in range(4):
        def zeroa(sl, a=accs[k]):
            a[sl] = jnp.zeros((L,), f32)
        _vmap_loop(zeroa, NPV)

    def outer(g, _):
        for b in range(2):
            j = 2 * g + b
            wait_in(b)
            if b == 0:
                start_in(j + 1, 1)
            else:
                @pl.when(g < NCH_B1 // 2 - 1)
                def _():
                    start_in(j + 1, 0)

            def inner(sl, b=b):
                si = srcs[b][sl]
                di = dsts[b][sl]
                al = plsc.load_gather(as_v, [si]) + plsc.load_gather(ad_v, [di])
                al = jnp.maximum(al, 0.2 * al)
                e = jnp.exp(al - bh)
                plsc.addupdate_scatter(den_v, [di], e)
                for k in range(2):
                    gi = plsc.load_gather(xpps[k], [si])
                    pb = plsc.bitcast(gi, jnp.bfloat16)
                    lo, hi = plsc.unpack(pb, format=plsc.PackFormat.INTERLEAVED,
                                         preferred_element_type=f32)
                    plsc.addupdate_scatter(accs[2 * k], [di], e * lo)
                    plsc.addupdate_scatter(accs[2 * k + 1], [di], e * hi)
            _vmap_loop(inner, CB1 // L, unroll=4)
        return 0
    lax.fori_loop(0, NCH_B1 // 2, outer, 0)
    for k in range(4):
        pltpu.sync_copy(accs[k], o1_hbm.at[hf, 4 * cq + k])
    pltpu.sync_copy(den_v, dp_hbm.at[h, q])


@functools.cache
def _sc_l1():
  return pl.kernel(
    _sc_l1_body,
    out_type=[
        jax.ShapeDtypeStruct((2, HH, NP), f32),
        jax.ShapeDtypeStruct((HEADS, 4, NP), f32),
    ],
    mesh=_mesh(),
    compiler_params=_SC_PARAMS,
    scratch_types=[
        pltpu.VMEM((NP,), f32),
        pltpu.VMEM((NP,), f32),
        pltpu.VMEM((NP,), f32),
        pltpu.VMEM((HEADS, 16), f32),
        pltpu.VMEM((NP,), f32),
        pltpu.VMEM((NP,), f32),
        pltpu.VMEM((NP,), f32),
        pltpu.VMEM((NP,), f32),
        pltpu.VMEM((NP,), i32),
        pltpu.VMEM((NP,), i32),
        pltpu.VMEM((CB1,), i32),
        pltpu.VMEM((CB1,), i32),
        pltpu.VMEM((CB1,), i32),
        pltpu.VMEM((CB1,), i32),
        pltpu.SemaphoreType.DMA,
        pltpu.SemaphoreType.DMA,
        pltpu.SemaphoreType.DMA,
        pltpu.SemaphoreType.DMA,
    ],
  )


def _sc_l2_body(s2_hbm, d2_hbm, hp_hbm, src_hbm, dst_hbm, bv_hbm,
                o2_hbm, dp_hbm,
                s2_v, d2_v, den_v, bv_v, acc0, acc1, hpp_v,
                src0, src1, dst0, dst1,
                ss0, ss1, sd0, sd1):
    wid = _wid()
    p = wid % 8
    qt = wid // 8
    accs = (acc0, acc1)
    srcs, dsts = (src0, src1), (dst0, dst1)
    sss, sds = (ss0, ss1), (sd0, sd1)
    base = qt * EQ

    def start_in(j, b):
        off = pl.multiple_of(base + j * CB2, 8)
        pltpu.async_copy(src_hbm.at[pl.ds(off, CB2)], srcs[b], sss[b])
        pltpu.async_copy(dst_hbm.at[pl.ds(off, CB2)], dsts[b], sds[b])

    def wait_in(b):
        pltpu.make_async_copy(src_hbm.at[pl.ds(0, CB2)], srcs[b], sss[b]).wait()
        pltpu.make_async_copy(dst_hbm.at[pl.ds(0, CB2)], dsts[b], sds[b]).wait()

    start_in(0, 0)

    pltpu.sync_copy(s2_hbm.at[0], s2_v)
    pltpu.sync_copy(d2_hbm.at[0], d2_v)
    pltpu.sync_copy(hp_hbm.at[p], hpp_v)
    pltpu.sync_copy(bv_hbm, bv_v)
    bh = bv_v[0, :]

    def zero(sl):
        den_v[sl] = jnp.zeros((L,), f32)
        acc0[sl] = jnp.zeros((L,), f32)
        acc1[sl] = jnp.zeros((L,), f32)
    _vmap_loop(zero, NPV)

    def outer(g, _):
        for b in range(2):
            j = 2 * g + b
            wait_in(b)
            if b == 0:
                start_in(j + 1, 1)
            else:
                @pl.when(g < NCH_B2 // 2 - 1)
                def _():
                    start_in(j + 1, 0)

            def inner(sl, b=b):
                si = srcs[b][sl]
                di = dsts[b][sl]
                al = plsc.load_gather(s2_v, [si]) + plsc.load_gather(d2_v, [di])
                al = jnp.maximum(al, 0.2 * al)
                e = jnp.exp(al - bh)
                plsc.addupdate_scatter(den_v, [di], e)
                gi = plsc.load_gather(hpp_v, [si])
                pb = plsc.bitcast(gi, jnp.bfloat16)
                lo, hi = plsc.unpack(pb, format=plsc.PackFormat.INTERLEAVED,
                                     preferred_element_type=f32)
                plsc.addupdate_scatter(acc0, [di], e * lo)
                plsc.addupdate_scatter(acc1, [di], e * hi)
            _vmap_loop(inner, CB2 // L, unroll=4)
        return 0
    lax.fori_loop(0, NCH_B2 // 2, outer, 0)
    pltpu.sync_copy(acc0, o2_hbm.at[qt, 2 * p])
    pltpu.sync_copy(acc1, o2_hbm.at[qt, 2 * p + 1])
    pltpu.sync_copy(den_v, dp_hbm.at[wid])


@functools.cache
def _sc_l2():
  return pl.kernel(
    _sc_l2_body,
    out_type=[
        jax.ShapeDtypeStruct((4, OUT, NP), f32),
        jax.ShapeDtypeStruct((NW, NP), f32),
    ],
    mesh=_mesh(),
    compiler_params=_SC_PARAMS,
    scratch_types=[
        pltpu.VMEM((NP,), f32),
        pltpu.VMEM((NP,), f32),
        pltpu.VMEM((NP,), f32),
        pltpu.VMEM((1, 16), f32),
        pltpu.VMEM((NP,), f32),
        pltpu.VMEM((NP,), f32),
        pltpu.VMEM((NP,), i32),
        pltpu.VMEM((CB2,), i32),
        pltpu.VMEM((CB2,), i32),
        pltpu.VMEM((CB2,), i32),
        pltpu.VMEM((CB2,), i32),
        pltpu.SemaphoreType.DMA,
        pltpu.SemaphoreType.DMA,
        pltpu.SemaphoreType.DMA,
        pltpu.SemaphoreType.DMA,
    ],
  )


def kernel(x, edge_index, W1, att_src1, att_dst1, b1, W2, att_src2, att_dst2, b2):
    x_p = jnp.pad(x, ((0, NP - N), (0, 0)))
    loop = jnp.arange(N, dtype=i32)
    padv = jnp.full((EP - ET,), N, i32)
    src_p = jnp.concatenate([edge_index[0].astype(i32), loop, padv])
    dst_p = jnp.concatenate([edge_index[1].astype(i32), loop, padv])

    eye = jnp.eye(HEADS, dtype=f32)
    asbd = (eye[:, :, None] * att_src1[None, :, :]).reshape(HEADS, HH)
    adbd = (eye[:, :, None] * att_dst1[None, :, :]).reshape(HEADS, HH)

    xpp, asT, adT, b1v = _tc1(x_p, W1[0::2], W1[1::2], asbd[:, 0::2],
                              asbd[:, 1::2], adbd[:, 0::2], adbd[:, 1::2])
    o1, dp1 = _sc_l1()(asT, adT, xpp, src_p, dst_p, b1v)
    hpp, s2, d2, b2v = _tc3(o1, dp1, b1.reshape(HH, 1), W2[0::2], W2[1::2],
                            att_src2[:, 0::2], att_src2[:, 1::2],
                            att_dst2[:, 0::2], att_dst2[:, 1::2])
    o2, dp2 = _sc_l2()(s2, d2, hpp, src_p, dst_p, b2v)
    (outf,) = _tc5(o2, dp2, b2.reshape(OUT, 1))
    return outf[:, :N].T


# speedup vs baseline: 1.0497x; 1.0030x over previous
import functools

import jax
import jax.numpy as jnp
from jax import lax
from jax.experimental import pallas as pl
from jax.experimental.pallas import tpu as pltpu
from jax.experimental.pallas import tpu_sc as plsc

N = 10000
F_IN = 128
HID = 8
HEADS = 8
HH = HEADS * HID
OUT = 16

NP = 10016
E0 = 320000
ET = E0 + N
EP = 344064
NC, NS, L = 2, 16, 16
NW = NC * NS
NPV = NP // L

CA1 = 2048
EQ1 = EP // 4
NCH_A1 = EQ1 // CA1
EH = EP // 2
CB1 = 4096
NCH_B1 = EH // CB1
EQ = EP // 4
CB2 = 2048
NCH_B2 = EQ // CB2
ES2 = EP // NW
CA2 = 1344
NCH_A2 = ES2 // CA2

f32 = jnp.float32
i32 = jnp.int32

_SC_PARAMS = pltpu.CompilerParams(needs_layout_passes=False)


@functools.cache
def _mesh():
    return plsc.VectorSubcoreMesh(core_axis_name="c", subcore_axis_name="s",
                                  num_cores=NC, num_subcores=NS)


def _wid():
    return lax.axis_index("s") * NC + lax.axis_index("c")


def _vmap_loop(ref_ops, n, unroll=4):
    @plsc.parallel_loop(0, n, unroll=unroll)
    def _(i):
        ref_ops(pl.ds(i * L, L))


def _tc1_body(x_ref, w1lo_ref, w1hi_ref, asbd_lo_ref, asbd_hi_ref,
              adbd_lo_ref, adbd_hi_ref,
              xpp_ref, asT_ref, adT_ref, b_ref):
    x = x_ref[...]
    dn = (((1,), (1,)), ((), ()))
    xpTlo = lax.dot_general(w1lo_ref[...], x, dn,
                            preferred_element_type=f32)
    xpThi = lax.dot_general(w1hi_ref[...], x, dn,
                            preferred_element_type=f32)
    lo_w = lax.bitcast_convert_type(xpTlo.astype(jnp.bfloat16),
                                    jnp.uint16).astype(jnp.uint32)
    hi_w = lax.bitcast_convert_type(xpThi.astype(jnp.bfloat16),
                                    jnp.uint16).astype(jnp.uint32)
    xpp_ref[...] = lax.bitcast_convert_type(lo_w | (hi_w << 16), jnp.int32)
    asT = (jnp.dot(asbd_lo_ref[...], xpTlo, preferred_element_type=f32)
           + jnp.dot(asbd_hi_ref[...], xpThi, preferred_element_type=f32))
    adT = (jnp.dot(adbd_lo_ref[...], xpTlo, preferred_element_type=f32)
           + jnp.dot(adbd_hi_ref[...], xpThi, preferred_element_type=f32))
    asT_ref[...] = asT
    adT_ref[...] = adT
    s = (jnp.max(asT, axis=1, keepdims=True)
         + jnp.max(adT, axis=1, keepdims=True))
    b = jnp.maximum(s, 0.2 * s)
    b_ref[...] = jnp.broadcast_to(b, (HEADS, 16))


_tc1 = pl.pallas_call(
    _tc1_body,
    out_shape=[
        jax.ShapeDtypeStruct((HH // 2, NP), i32),
        jax.ShapeDtypeStruct((HEADS, NP), f32),
        jax.ShapeDtypeStruct((HEADS, NP), f32),
        jax.ShapeDtypeStruct((HEADS, 16), f32),
    ],
)


def _tc3_body(o1_ref, dp_ref, b1_ref, w2lo_ref, w2hi_ref, as2wlo_ref,
              as2whi_ref, ad2wlo_ref, ad2whi_ref,
              hpp_ref, s2_ref, d2_ref, b2_ref):
    den = 0.5 * jnp.sum(dp_ref[...], axis=1)
    rden = 1.0 / (den + 1e-16)
    rden64 = jnp.broadcast_to(rden[:, None, :], (HEADS, HID, NP)).reshape(HH, NP)
    hsum = (o1_ref[0] + o1_ref[1]) * rden64 + b1_ref[...]
    h = jnp.where(hsum > 0, hsum, jnp.exp(jnp.minimum(hsum, 0.0)) - 1.0)
    hpTlo = jnp.dot(w2lo_ref[...], h, preferred_element_type=f32)
    hpThi = jnp.dot(w2hi_ref[...], h, preferred_element_type=f32)
    lo_w = lax.bitcast_convert_type(hpTlo.astype(jnp.bfloat16),
                                    jnp.uint16).astype(jnp.uint32)
    hi_w = lax.bitcast_convert_type(hpThi.astype(jnp.bfloat16),
                                    jnp.uint16).astype(jnp.uint32)
    hpp_ref[...] = lax.bitcast_convert_type(lo_w | (hi_w << 16), jnp.int32)
    s2 = (jnp.dot(as2wlo_ref[...], hpTlo, preferred_element_type=f32)
          + jnp.dot(as2whi_ref[...], hpThi, preferred_element_type=f32))
    d2 = (jnp.dot(ad2wlo_ref[...], hpTlo, preferred_element_type=f32)
          + jnp.dot(ad2whi_ref[...], hpThi, preferred_element_type=f32))
    s2_ref[...] = s2
    d2_ref[...] = d2
    t = (jnp.max(s2, axis=1, keepdims=True)
         + jnp.max(d2, axis=1, keepdims=True))
    b2 = jnp.maximum(t, 0.2 * t)
    b2_ref[...] = jnp.broadcast_to(b2, (1, 16))


_tc3 = pl.pallas_call(
    _tc3_body,
    out_shape=[
        jax.ShapeDtypeStruct((OUT // 2, NP), i32),
        jax.ShapeDtypeStruct((1, NP), f32),
        jax.ShapeDtypeStruct((1, NP), f32),
        jax.ShapeDtypeStruct((1, 16), f32),
    ],
)


def _tc5_body(o2_ref, dp_ref, b2_ref, out_ref):
    den = 0.125 * jnp.sum(dp_ref[...], axis=0, keepdims=True)
    rden = 1.0 / (den + 1e-16)
    osum = o2_ref[0] + o2_ref[1] + o2_ref[2] + o2_ref[3]
    out_ref[...] = osum * rden + b2_ref[...]


_tc5 = pl.pallas_call(
    _tc5_body,
    out_shape=[jax.ShapeDtypeStruct((OUT, NP), f32)],
)


def _sc_l1_body(as_hbm, ad_hbm, xp_hbm, src_hbm, dst_hbm, bv_hbm,
                o1_hbm, dp_hbm,
                as_v, ad_v, den_v, bv_v,
                acc0, acc1, acc2, acc3, xp0, xp1,
                src0, src1, dst0, dst1,
                ss0, ss1, sd0, sd1):
    wid = _wid()
    cq = wid % 16
    hf = wid // 16
    h = cq // 2
    q = (cq % 2) * 2 + hf
    accs = (acc0, acc1, acc2, acc3)
    xpps = (xp0, xp1)
    srcs, dsts = (src0, src1), (dst0, dst1)
    sss, sds = (ss0, ss1), (sd0, sd1)
    base = hf * EH

    def start_in(j, b):
        off = pl.multiple_of(base + j * CB1, 8)
        pltpu.async_copy(src_hbm.at[pl.ds(off, CB1)], srcs[b], sss[b])
        pltpu.async_copy(dst_hbm.at[pl.ds(off, CB1)], dsts[b], sds[b])

    def wait_in(b):
        pltpu.make_async_copy(src_hbm.at[pl.ds(0, CB1)], srcs[b], sss[b]).wait()
        pltpu.make_async_copy(dst_hbm.at[pl.ds(0, CB1)], dsts[b], sds[b]).wait()

    start_in(0, 0)

    pltpu.sync_copy(as_hbm.at[h], as_v)
    pltpu.sync_copy(ad_hbm.at[h], ad_v)
    pltpu.sync_copy(bv_hbm, bv_v)
    bh = bv_v[h, :]

    def zero(sl):
        den_v[sl] = jnp.zeros((L,), f32)
    _vmap_loop(zero, NPV)

    for k in range(2):
        pltpu.sync_copy(xp_hbm.at[2 * cq + k], xpps[k])
    for k in range(4):
        def zeroa(sl, a=accs[k]):
            a[sl] = jnp.zeros((L,), f32)
        _vmap_loop(zeroa, NPV)

    def outer(g, _):
        for b in range(2):
            j = 2 * g + b
            wait_in(b)
            if b == 0:
                start_in(j + 1, 1)
            else:
                @pl.when(g < NCH_B1 // 2 - 1)
                def _():
                    start_in(j + 1, 0)

            def inner(sl, b=b):
                si = srcs[b][sl]
                di = dsts[b][sl]
                al = plsc.load_gather(as_v, [si]) + plsc.load_gather(ad_v, [di])
                al = jnp.maximum(al, 0.2 * al)
                e = jnp.exp(al - bh)
                plsc.addupdate_scatter(den_v, [di], e)
                for k in range(2):
                    gi = plsc.load_gather(xpps[k], [si])
                    pb = plsc.bitcast(gi, jnp.bfloat16)
                    lo, hi = plsc.unpack(pb, format=plsc.PackFormat.INTERLEAVED,
                                         preferred_element_type=f32)
                    plsc.addupdate_scatter(accs[2 * k], [di], e * lo)
                    plsc.addupdate_scatter(accs[2 * k + 1], [di], e * hi)
            _vmap_loop(inner, CB1 // L, unroll=4)
        return 0
    lax.fori_loop(0, NCH_B1 // 2, outer, 0)
    for k in range(4):
        pltpu.sync_copy(accs[k], o1_hbm.at[hf, 4 * cq + k])
    pltpu.sync_copy(den_v, dp_hbm.at[h, q])


@functools.cache
def _sc_l1():
  return pl.kernel(
    _sc_l1_body,
    out_type=[
        jax.ShapeDtypeStruct((2, HH, NP), f32),
        jax.ShapeDtypeStruct((HEADS, 4, NP), f32),
    ],
    mesh=_mesh(),
    compiler_params=_SC_PARAMS,
    scratch_types=[
        pltpu.VMEM((NP,), f32),
        pltpu.VMEM((NP,), f32),
        pltpu.VMEM((NP,), f32),
        pltpu.VMEM((HEADS, 16), f32),
        pltpu.VMEM((NP,), f32),
        pltpu.VMEM((NP,), f32),
        pltpu.VMEM((NP,), f32),
        pltpu.VMEM((NP,), f32),
        pltpu.VMEM((NP,), i32),
        pltpu.VMEM((NP,), i32),
        pltpu.VMEM((CB1,), i32),
        pltpu.VMEM((CB1,), i32),
        pltpu.VMEM((CB1,), i32),
        pltpu.VMEM((CB1,), i32),
        pltpu.SemaphoreType.DMA,
        pltpu.SemaphoreType.DMA,
        pltpu.SemaphoreType.DMA,
        pltpu.SemaphoreType.DMA,
    ],
  )


def _sc_l2_body(s2_hbm, d2_hbm, hp_hbm, src_hbm, dst_hbm, bv_hbm,
                o2_hbm, dp_hbm,
                s2_v, d2_v, den_v, bv_v, acc0, acc1, hpp_v,
                src0, src1, dst0, dst1,
                ss0, ss1, sd0, sd1):
    wid = _wid()
    p = wid % 8
    qt = wid // 8
    accs = (acc0, acc1)
    srcs, dsts = (src0, src1), (dst0, dst1)
    sss, sds = (ss0, ss1), (sd0, sd1)
    base = qt * EQ

    def start_in(j, b):
        off = pl.multiple_of(base + j * CB2, 8)
        pltpu.async_copy(src_hbm.at[pl.ds(off, CB2)], srcs[b], sss[b])
        pltpu.async_copy(dst_hbm.at[pl.ds(off, CB2)], dsts[b], sds[b])

    def wait_in(b):
        pltpu.make_async_copy(src_hbm.at[pl.ds(0, CB2)], srcs[b], sss[b]).wait()
        pltpu.make_async_copy(dst_hbm.at[pl.ds(0, CB2)], dsts[b], sds[b]).wait()

    start_in(0, 0)

    pltpu.sync_copy(s2_hbm.at[0], s2_v)
    pltpu.sync_copy(d2_hbm.at[0], d2_v)
    pltpu.sync_copy(hp_hbm.at[p], hpp_v)
    pltpu.sync_copy(bv_hbm, bv_v)
    bh = bv_v[0, :]

    def zero(sl):
        den_v[sl] = jnp.zeros((L,), f32)
        acc0[sl] = jnp.zeros((L,), f32)
        acc1[sl] = jnp.zeros((L,), f32)
    _vmap_loop(zero, NPV)

    def outer(g, _):
        for b in range(2):
            j = 2 * g + b
            wait_in(b)
            if b == 0:
                start_in(j + 1, 1)
            else:
                @pl.when(g < NCH_B2 // 2 - 1)
                def _():
                    start_in(j + 1, 0)

            def inner(sl, b=b):
                si = srcs[b][sl]
                di = dsts[b][sl]
                al = plsc.load_gather(s2_v, [si]) + plsc.load_gather(d2_v, [di])
                al = jnp.maximum(al, 0.2 * al)
                e = jnp.exp(al - bh)
                plsc.addupdate_scatter(den_v, [di], e)
                gi = plsc.load_gather(hpp_v, [si])
                pb = plsc.bitcast(gi, jnp.bfloat16)
                lo, hi = plsc.unpack(pb, format=plsc.PackFormat.INTERLEAVED,
                                     preferred_element_type=f32)
                plsc.addupdate_scatter(acc0, [di], e * lo)
                plsc.addupdate_scatter(acc1, [di], e * hi)
            _vmap_loop(inner, CB2 // L, unroll=4)
        return 0
    lax.fori_loop(0, NCH_B2 // 2, outer, 0)
    pltpu.sync_copy(acc0, o2_hbm.at[qt, 2 * p])
    pltpu.sync_copy(acc1, o2_hbm.at[qt, 2 * p + 1])
    pltpu.sync_copy(den_v, dp_hbm.at[wid])


@functools.cache
def _sc_l2():
  return pl.kernel(
    _sc_l2_body,
    out_type=[
        jax.ShapeDtypeStruct((4, OUT, NP), f32),
        jax.ShapeDtypeStruct((NW, NP), f32),
    ],
    mesh=_mesh(),
    compiler_params=_SC_PARAMS,
    scratch_types=[
        pltpu.VMEM((NP,), f32),
        pltpu.VMEM((NP,), f32),
        pltpu.VMEM((NP,), f32),
        pltpu.VMEM((1, 16), f32),
        pltpu.VMEM((NP,), f32),
        pltpu.VMEM((NP,), f32),
        pltpu.VMEM((NP,), i32),
        pltpu.VMEM((CB2,), i32),
        pltpu.VMEM((CB2,), i32),
        pltpu.VMEM((CB2,), i32),
        pltpu.VMEM((CB2,), i32),
        pltpu.SemaphoreType.DMA,
        pltpu.SemaphoreType.DMA,
        pltpu.SemaphoreType.DMA,
        pltpu.SemaphoreType.DMA,
    ],
  )


def kernel(x, edge_index, W1, att_src1, att_dst1, b1, W2, att_src2, att_dst2, b2):
    x_p = jnp.pad(x, ((0, NP - N), (0, 0)))
    loop = jnp.arange(N, dtype=i32)
    padv = jnp.full((EP - ET,), N, i32)
    src_p = jnp.concatenate([edge_index[0].astype(i32), loop, padv])
    dst_p = jnp.concatenate([edge_index[1].astype(i32), loop, padv])

    eye = jnp.eye(HEADS, dtype=f32)
    asbd = (eye[:, :, None] * att_src1[None, :, :]).reshape(HEADS, HH)
    adbd = (eye[:, :, None] * att_dst1[None, :, :]).reshape(HEADS, HH)

    xpp, asT, adT, b1v = _tc1(x_p, W1[0::2], W1[1::2], asbd[:, 0::2],
                              asbd[:, 1::2], adbd[:, 0::2], adbd[:, 1::2])
    o1, dp1 = _sc_l1()(asT, adT, xpp, src_p, dst_p, b1v)
    hpp, s2, d2, b2v = _tc3(o1, dp1, b1.reshape(HH, 1), W2[0::2], W2[1::2],
                            att_src2[:, 0::2], att_src2[:, 1::2],
                            att_dst2[:, 0::2], att_dst2[:, 1::2])
    o2, dp2 = _sc_l2()(s2, d2, hpp, src_p, dst_p, b2v)
    (outf,) = _tc5(o2, dp2, b2.reshape(OUT, 1))
    return outf[:, :N].T
